# Initial kernel scaffold; baseline (speedup 1.0000x reference)
import jax
import jax.numpy as jnp
from jax.experimental import pallas as pl


def kernel(user_id, book_id, author_label, category_label, publisher_label, page_count, average_rating, ratings_count, published_year, full_text_embeddings, W_wide, b_wide, user_table, book_table, author_table, category_table, publisher_table, W1, b1, W2, b2, W3, b3):
    raise NotImplementedError("write your pallas kernel here")


# speedup vs baseline: 2.2572x; 2.2572x over previous
import functools

import jax
import jax.numpy as jnp
from jax import lax
from jax.experimental import pallas as pl
from jax.experimental.pallas import tpu as pltpu
from jax.experimental.pallas import tpu_sc as plsc

B = 16384
D = 32
TEXT = 768
NCAT = 20
CAT_ROWS = 1000

NC, NS = 2, 16
NW = NC * NS
BPW = B // NW
ICH = 128
NCHUNK = BPW // ICH
LANES = 16
CGRP = 2


def _gather_body(uid, bid, aid, pid, clbl, utab, btab, atab, ptab, ctab,
                 out_u, out_b, out_a, out_c, out_p,
                 uidx, bidx, aidx, pidx,
                 urows, brows, arows, prows,
                 cidx, ctab_v, cbuf, gsem):
    wid = lax.axis_index("s") * NC + lax.axis_index("c")
    base = wid * BPW

    pltpu.sync_copy(uid.at[wid], uidx)
    pltpu.sync_copy(bid.at[wid], bidx)
    pltpu.sync_copy(aid.at[wid], aidx)
    pltpu.sync_copy(pid.at[wid], pidx)
    pltpu.sync_copy(clbl.at[wid], cidx)
    pltpu.sync_copy(ctab, ctab_v)

    copies = []
    for idx_v, rows_v, tab in ((uidx, urows, utab), (bidx, brows, btab),
                               (aidx, arows, atab), (pidx, prows, ptab)):
        for j in range(NCHUNK):
            copies.append(pltpu.async_copy(
                tab.at[idx_v.at[j]], rows_v.at[pl.ds(j * ICH, ICH)], gsem))

    lane = lax.iota(jnp.int32, LANES)
    inv = jnp.full((LANES,), 1.0 / NCAT, jnp.float32)

    def cat_chunk(k, carry):
        e0 = pl.multiple_of(k * LANES, LANES)
        for g in range(CGRP):
            def j_body(j, acc):
                row = cidx[j, pl.ds(e0, LANES)]
                return tuple(
                    acc[c] + plsc.load_gather(
                        ctab_v,
                        [row, jnp.full((LANES,), g * LANES + c, jnp.int32)])
                    for c in range(LANES))
            acc = lax.fori_loop(
                0, NCAT, j_body,
                tuple(jnp.zeros((LANES,), jnp.float32) for _ in range(LANES)))
            rows16 = e0 + lane
            for c in range(LANES):
                plsc.store_scatter(
                    cbuf,
                    [rows16, jnp.full((LANES,), g * LANES + c, jnp.int32)],
                    acc[c] * inv)
        return carry

    lax.fori_loop(0, BPW // LANES, cat_chunk, 0)

    for cp in copies:
        cp.wait()

    pltpu.sync_copy(urows, out_u.at[pl.ds(base, BPW)])
    pltpu.sync_copy(brows, out_b.at[pl.ds(base, BPW)])
    pltpu.sync_copy(arows, out_a.at[pl.ds(base, BPW)])
    pltpu.sync_copy(cbuf, out_c.at[pl.ds(base, BPW)])
    pltpu.sync_copy(prows, out_p.at[pl.ds(base, BPW)])


_gather5 = pl.kernel(
    _gather_body,
    out_type=tuple(jax.ShapeDtypeStruct((B, D), jnp.float32)
                   for _ in range(5)),
    mesh=plsc.VectorSubcoreMesh(core_axis_name="c", subcore_axis_name="s"),
    scratch_types=[
        pltpu.VMEM((NCHUNK, ICH), jnp.int32),
        pltpu.VMEM((NCHUNK, ICH), jnp.int32),
        pltpu.VMEM((NCHUNK, ICH), jnp.int32),
        pltpu.VMEM((NCHUNK, ICH), jnp.int32),
        pltpu.VMEM((BPW, D), jnp.float32),
        pltpu.VMEM((BPW, D), jnp.float32),
        pltpu.VMEM((BPW, D), jnp.float32),
        pltpu.VMEM((BPW, D), jnp.float32),
        pltpu.VMEM((NCAT, BPW), jnp.int32),
        pltpu.VMEM((CAT_ROWS, D), jnp.float32),
        pltpu.VMEM((BPW, D), jnp.float32),
        pltpu.SemaphoreType.DMA,
    ],
    compiler_params=pltpu.CompilerParams(needs_layout_passes=False,
                                         use_tc_tiling_on_sc=False),
)


BB = 512


def _mlp_body(eu_ref, eb_ref, ea_ref, ec_ref, ep_ref, wf_ref, text_ref,
              w1u_ref, w1b_ref, w1a_ref, w1c_ref, w1p_ref, w1w_ref, w1t_ref,
              b1_ref, w2_ref, b2_ref, w3_ref, ww_ref, out_ref):
    wf = wf_ref[...]
    h = jnp.dot(text_ref[...], w1t_ref[...],
                preferred_element_type=jnp.float32)
    for e_ref, w_ref in ((eu_ref, w1u_ref), (eb_ref, w1b_ref),
                         (ea_ref, w1a_ref), (ec_ref, w1c_ref),
                         (ep_ref, w1p_ref)):
        h = h + jnp.dot(e_ref[...], w_ref[...],
                        preferred_element_type=jnp.float32)
    h = h + jnp.dot(wf, w1w_ref[...], preferred_element_type=jnp.float32)
    h = jnp.maximum(h + b1_ref[...], 0.0)
    h2 = jnp.dot(h, w2_ref[...], preferred_element_type=jnp.float32)
    h2 = jnp.maximum(h2 + b2_ref[...], 0.0)
    deep = jnp.sum(h2 * w3_ref[...], axis=1)
    wide = jnp.sum(wf * ww_ref[...], axis=1)
    out_ref[...] = deep + wide


_mlp = pl.pallas_call(
    _mlp_body,
    grid=(B // BB,),
    in_specs=[
        pl.BlockSpec((BB, D), lambda i: (i, 0)),
        pl.BlockSpec((BB, D), lambda i: (i, 0)),
        pl.BlockSpec((BB, D), lambda i: (i, 0)),
        pl.BlockSpec((BB, D), lambda i: (i, 0)),
        pl.BlockSpec((BB, D), lambda i: (i, 0)),
        pl.BlockSpec((BB, 4), lambda i: (i, 0)),
        pl.BlockSpec((BB, TEXT), lambda i: (i, 0)),
        pl.BlockSpec((D, 128), lambda i: (0, 0)),
        pl.BlockSpec((D, 128), lambda i: (0, 0)),
        pl.BlockSpec((D, 128), lambda i: (0, 0)),
        pl.BlockSpec((D, 128), lambda i: (0, 0)),
        pl.BlockSpec((D, 128), lambda i: (0, 0)),
        pl.BlockSpec((4, 128), lambda i: (0, 0)),
        pl.BlockSpec((TEXT, 128), lambda i: (0, 0)),
        pl.BlockSpec((1, 128), lambda i: (0, 0)),
        pl.BlockSpec((128, 64), lambda i: (0, 0)),
        pl.BlockSpec((1, 64), lambda i: (0, 0)),
        pl.BlockSpec((1, 64), lambda i: (0, 0)),
        pl.BlockSpec((1, 4), lambda i: (0, 0)),
    ],
    out_specs=pl.BlockSpec((BB,), lambda i: (i,)),
    out_shape=jax.ShapeDtypeStruct((B,), jnp.float32),
    compiler_params=pltpu.CompilerParams(
        dimension_semantics=("arbitrary",)),
)


def kernel(user_id, book_id, author_label, category_label, publisher_label,
           page_count, average_rating, ratings_count, published_year,
           full_text_embeddings, W_wide, b_wide, user_table, book_table,
           author_table, category_table, publisher_table, W1, b1, W2, b2,
           W3, b3):
    i32 = jnp.int32
    uid = user_id.astype(i32).reshape(NW, NCHUNK, ICH)
    bid = book_id.astype(i32).reshape(NW, NCHUNK, ICH)
    aid = author_label.astype(i32).reshape(NW, NCHUNK, ICH)
    pid = publisher_label.astype(i32).reshape(NW, NCHUNK, ICH)
    clbl = (category_label.astype(i32)
            .reshape(NW, BPW, NCAT).transpose(0, 2, 1))

    eu, eb, ea, ec, ep = _gather5(uid, bid, aid, pid, clbl, user_table,
                                  book_table, author_table, publisher_table,
                                  category_table)

    wf = jnp.stack([page_count, average_rating, ratings_count,
                    published_year], axis=1)
    out = _mlp(eu, eb, ea, ec, ep, wf, full_text_embeddings,
               W1[:, 0 * D:1 * D].T, W1[:, 1 * D:2 * D].T,
               W1[:, 2 * D:3 * D].T, W1[:, 3 * D:4 * D].T,
               W1[:, 4 * D:5 * D].T, W1[:, 5 * D:5 * D + 4].T,
               W1[:, 5 * D + 4:].T, b1.reshape(1, 128),
               W2.T, b2.reshape(1, 64), W3, W_wide)
    return out + (b3[0] + b_wide[0])
